# baseline (speedup 1.0000x reference)
import functools

import jax
import jax.numpy as jnp
from jax import lax
from jax.experimental import pallas as pl
from jax.experimental.pallas import tpu as pltpu
from jax.experimental.pallas import tpu_sc as plsc

N_USERS = 5000
N_ITEMS = 5000
N = N_USERS + N_ITEMS
D = 16
EPS = 1e-12

BM = 400
NM = N // BM


def _gcn_body(a_ref, u_ref, i_ref, out_ref, pad_ref, c1_ref, e0_ref):
    l = pl.program_id(0)
    m = pl.program_id(1)
    row = pl.ds(m * BM, BM)

    @pl.when((l == 0) & (m == 0))
    def _():
        e0_ref[0:N_USERS, :] = u_ref[...]
        e0_ref[N_USERS:N, :] = i_ref[...]

    cur = jnp.where(l == 0, e0_ref[...], c1_ref[...])
    acc = jnp.dot(a_ref[...], cur, preferred_element_type=jnp.float32)
    nrm = jnp.sqrt(jnp.sum(acc * acc, axis=1, keepdims=True))
    c = acc / jnp.maximum(nrm, EPS)

    @pl.when(l == 0)
    def _():
        c1_ref[row, :] = c
        out_ref[...] = c

    @pl.when(l == 1)
    def _():
        mean = (e0_ref[row, :] + c1_ref[row, :] + c) * (1.0 / 3.0)
        out_ref[...] = mean
        pad_ref[:, 0:D] = mean


def _propagate(a, u, i, interpret=False):
    return pl.pallas_call(
        _gcn_body,
        grid=(2, NM),
        in_specs=[
            pl.BlockSpec((BM, N), lambda l, m: (m, 0)),
            pl.BlockSpec((N_USERS, D), lambda l, m: (0, 0)),
            pl.BlockSpec((N_ITEMS, D), lambda l, m: (0, 0)),
        ],
        out_specs=[
            pl.BlockSpec((BM, D), lambda l, m: (m, 0)),
            pl.BlockSpec((BM, 128), lambda l, m: (m, 0)),
        ],
        out_shape=[
            jax.ShapeDtypeStruct((N, D), jnp.float32),
            jax.ShapeDtypeStruct((N, 128), jnp.float32),
        ],
        scratch_shapes=[
            pltpu.VMEM((N, D), jnp.float32),
            pltpu.VMEM((N, D), jnp.float32),
        ],
        interpret=interpret,
    )(a, u, i)


B = 4096
NW = 32
BPW = B // NW

_SC_MESH = plsc.VectorSubcoreMesh(core_axis_name="c", subcore_axis_name="s")


@functools.partial(
    pl.kernel,
    mesh=_SC_MESH,
    out_type=jax.ShapeDtypeStruct((3 * B, 128), jnp.float32),
    scratch_types=[
        pltpu.VMEM((BPW,), jnp.int32),
        pltpu.VMEM((BPW,), jnp.int32),
        pltpu.VMEM((BPW,), jnp.int32),
        pltpu.VMEM((BPW, 128), jnp.float32),
        pltpu.VMEM((BPW, 128), jnp.float32),
        pltpu.VMEM((BPW, 128), jnp.float32),
        pltpu.SemaphoreType.DMA,
        pltpu.SemaphoreType.DMA,
    ],
)
def _sc_gather(emb_hbm, uid_hbm, pid_hbm, nid_hbm, out_hbm,
               uidx, pidx, nidx, urows, prows, nrows, sem, wsem):
    wid = lax.axis_index("s") * 2 + lax.axis_index("c")
    base = wid * BPW
    iu = pltpu.async_copy(uid_hbm.at[pl.ds(base, BPW)], uidx, wsem)
    ip = pltpu.async_copy(pid_hbm.at[pl.ds(base, BPW)], pidx, wsem)
    inn = pltpu.async_copy(nid_hbm.at[pl.ds(base, BPW)], nidx, wsem)
    iu.wait()
    cu = pltpu.async_copy(emb_hbm.at[uidx], urows, sem)
    ip.wait()
    inn.wait()
    for j in range(BPW // 16):
        sl = pl.ds(j * 16, 16)
        pidx[sl] = pidx[sl] + N_USERS
        nidx[sl] = nidx[sl] + N_USERS
    cp = pltpu.async_copy(emb_hbm.at[pidx], prows, sem)
    cn = pltpu.async_copy(emb_hbm.at[nidx], nrows, sem)
    cu.wait()
    wu = pltpu.async_copy(urows, out_hbm.at[pl.ds(base, BPW)], wsem)
    cp.wait()
    wp = pltpu.async_copy(prows, out_hbm.at[pl.ds(B + base, BPW)], wsem)
    cn.wait()
    wn = pltpu.async_copy(nrows, out_hbm.at[pl.ds(2 * B + base, BPW)], wsem)
    wu.wait()
    wp.wait()
    wn.wait()


NC = 4
CB = B // NC


def _loss_body(u_ref, p_ref, n_ref, out_ref, acc_ref):
    i = pl.program_id(0)
    u = u_ref[:, 0:D]
    diff = (jnp.sum(u * p_ref[:, 0:D], axis=1)
            - jnp.sum(u * n_ref[:, 0:D], axis=1))
    part = jnp.sum(jnp.log(jax.nn.sigmoid(diff)))

    @pl.when(i == 0)
    def _():
        acc_ref[0] = part

    @pl.when(i > 0)
    def _():
        acc_ref[0] += part

    @pl.when(i == NC - 1)
    def _():
        out_ref[0, 0] = -acc_ref[0] * (1.0 / B)


def _loss(rows):
    out = pl.pallas_call(
        _loss_body,
        grid=(NC,),
        in_specs=[
            pl.BlockSpec((CB, 128), lambda i: (i, 0)),
            pl.BlockSpec((CB, 128), lambda i: (i + NC, 0)),
            pl.BlockSpec((CB, 128), lambda i: (i + 2 * NC, 0)),
        ],
        out_specs=pl.BlockSpec(memory_space=pltpu.SMEM),
        out_shape=jax.ShapeDtypeStruct((1, 1), jnp.float32),
        scratch_shapes=[pltpu.SMEM((1,), jnp.float32)],
    )(rows, rows, rows)
    return out[0, 0]


def kernel(user_emb, item_emb, edge_index, user_id, pos_item, neg_item):
    all_emb, pad = _propagate(edge_index, user_emb, item_emb)
    rows = _sc_gather(pad, user_id, pos_item, neg_item)
    rec_loss = _loss(rows)
    return (rec_loss, all_emb)


# speedup vs baseline: 1.0259x; 1.0259x over previous
import functools

import jax
import jax.numpy as jnp
from jax import lax
from jax.experimental import pallas as pl
from jax.experimental.pallas import tpu as pltpu
from jax.experimental.pallas import tpu_sc as plsc

N_USERS = 5000
N_ITEMS = 5000
N = N_USERS + N_ITEMS
D = 16
EPS = 1e-12

BM = 400
NM = N // BM


def _gcn_body(a_ref, ut_ref, it_ref, out_ref, pad_ref, c1_ref, e0_ref):
    l = pl.program_id(0)
    m = pl.program_id(1)
    row = pl.ds(m * BM, BM)

    @pl.when((l == 0) & (m == 0))
    def _():
        eye = jnp.eye(D, dtype=jnp.float32)
        e0_ref[0:N_USERS, :] = lax.dot_general(
            ut_ref[...], eye, (((0,), (0,)), ((), ())),
            preferred_element_type=jnp.float32)
        e0_ref[N_USERS:N, :] = lax.dot_general(
            it_ref[...], eye, (((0,), (0,)), ((), ())),
            preferred_element_type=jnp.float32)

    cur = jnp.where(l == 0, e0_ref[...], c1_ref[...])
    acc = jnp.dot(a_ref[...], cur, preferred_element_type=jnp.float32)
    nrm = jnp.sqrt(jnp.sum(acc * acc, axis=1, keepdims=True))
    c = acc / jnp.maximum(nrm, EPS)

    @pl.when(l == 0)
    def _():
        c1_ref[row, :] = c
        out_ref[...] = c

    @pl.when(l == 1)
    def _():
        mean = (e0_ref[row, :] + c1_ref[row, :] + c) * (1.0 / 3.0)
        out_ref[...] = mean
        pad_ref[:, 0:D] = mean


def _propagate(a, ut, it, interpret=False):
    return pl.pallas_call(
        _gcn_body,
        grid=(2, NM),
        in_specs=[
            pl.BlockSpec((BM, N), lambda l, m: (m, 0)),
            pl.BlockSpec((D, N_USERS), lambda l, m: (0, 0)),
            pl.BlockSpec((D, N_ITEMS), lambda l, m: (0, 0)),
        ],
        out_specs=[
            pl.BlockSpec((BM, D), lambda l, m: (m, 0)),
            pl.BlockSpec((BM, 128), lambda l, m: (m, 0)),
        ],
        out_shape=[
            jax.ShapeDtypeStruct((N, D), jnp.float32),
            jax.ShapeDtypeStruct((N, 128), jnp.float32),
        ],
        scratch_shapes=[
            pltpu.VMEM((N, D), jnp.float32),
            pltpu.VMEM((N, D), jnp.float32),
        ],
        interpret=interpret,
    )(a, ut, it)


B = 4096
NW = 32
BPW = B // NW

_SC_MESH = plsc.VectorSubcoreMesh(core_axis_name="c", subcore_axis_name="s")


@functools.partial(
    pl.kernel,
    mesh=_SC_MESH,
    out_type=jax.ShapeDtypeStruct((3 * B, 128), jnp.float32),
    scratch_types=[
        pltpu.VMEM((BPW,), jnp.int32),
        pltpu.VMEM((BPW,), jnp.int32),
        pltpu.VMEM((BPW,), jnp.int32),
        pltpu.VMEM((BPW, 128), jnp.float32),
        pltpu.VMEM((BPW, 128), jnp.float32),
        pltpu.VMEM((BPW, 128), jnp.float32),
        pltpu.SemaphoreType.DMA,
        pltpu.SemaphoreType.DMA,
    ],
)
def _sc_gather(emb_hbm, uid_hbm, pid_hbm, nid_hbm, out_hbm,
               uidx, pidx, nidx, urows, prows, nrows, sem, wsem):
    wid = lax.axis_index("s") * 2 + lax.axis_index("c")
    base = wid * BPW
    iu = pltpu.async_copy(uid_hbm.at[pl.ds(base, BPW)], uidx, wsem)
    ip = pltpu.async_copy(pid_hbm.at[pl.ds(base, BPW)], pidx, wsem)
    inn = pltpu.async_copy(nid_hbm.at[pl.ds(base, BPW)], nidx, wsem)
    iu.wait()
    cu = pltpu.async_copy(emb_hbm.at[uidx], urows, sem)
    ip.wait()
    inn.wait()
    for j in range(BPW // 16):
        sl = pl.ds(j * 16, 16)
        pidx[sl] = pidx[sl] + N_USERS
        nidx[sl] = nidx[sl] + N_USERS
    cp = pltpu.async_copy(emb_hbm.at[pidx], prows, sem)
    cn = pltpu.async_copy(emb_hbm.at[nidx], nrows, sem)
    cu.wait()
    wu = pltpu.async_copy(urows, out_hbm.at[pl.ds(base, BPW)], wsem)
    cp.wait()
    wp = pltpu.async_copy(prows, out_hbm.at[pl.ds(B + base, BPW)], wsem)
    cn.wait()
    wn = pltpu.async_copy(nrows, out_hbm.at[pl.ds(2 * B + base, BPW)], wsem)
    wu.wait()
    wp.wait()
    wn.wait()


NC = 4
CB = B // NC


def _loss_body(u_ref, p_ref, n_ref, out_ref, acc_ref):
    i = pl.program_id(0)
    u = u_ref[:, 0:D]
    diff = (jnp.sum(u * p_ref[:, 0:D], axis=1)
            - jnp.sum(u * n_ref[:, 0:D], axis=1))
    part = jnp.sum(jnp.log(jax.nn.sigmoid(diff)))

    @pl.when(i == 0)
    def _():
        acc_ref[0] = part

    @pl.when(i > 0)
    def _():
        acc_ref[0] += part

    @pl.when(i == NC - 1)
    def _():
        out_ref[0, 0] = -acc_ref[0] * (1.0 / B)


def _loss(rows):
    out = pl.pallas_call(
        _loss_body,
        grid=(NC,),
        in_specs=[
            pl.BlockSpec((CB, 128), lambda i: (i, 0)),
            pl.BlockSpec((CB, 128), lambda i: (i + NC, 0)),
            pl.BlockSpec((CB, 128), lambda i: (i + 2 * NC, 0)),
        ],
        out_specs=pl.BlockSpec(memory_space=pltpu.SMEM),
        out_shape=jax.ShapeDtypeStruct((1, 1), jnp.float32),
        scratch_shapes=[pltpu.SMEM((1,), jnp.float32)],
    )(rows, rows, rows)
    return out[0, 0]


def kernel(user_emb, item_emb, edge_index, user_id, pos_item, neg_item):
    all_emb, pad = _propagate(edge_index, user_emb.T, item_emb.T)
    rows = _sc_gather(pad, user_id, pos_item, neg_item)
    rec_loss = _loss(rows)
    return (rec_loss, all_emb)


# speedup vs baseline: 1.0333x; 1.0073x over previous
import functools

import jax
import jax.numpy as jnp
from jax import lax
from jax.experimental import pallas as pl
from jax.experimental.pallas import tpu as pltpu
from jax.experimental.pallas import tpu_sc as plsc

N_USERS = 5000
N_ITEMS = 5000
N = N_USERS + N_ITEMS
D = 16
EPS = 1e-12

BM = 400
NM = N // BM


def _gcn_body(a_ref, ut_ref, it_ref, out_ref, pad_ref, c1_ref, e0_ref, mean_ref):
    l = pl.program_id(0)
    m = pl.program_id(1)
    row = pl.ds(m * BM, BM)

    @pl.when((l == 0) & (m == 0))
    def _():
        eye = jnp.eye(D, dtype=jnp.float32)
        e0_ref[0:N_USERS, :] = lax.dot_general(
            ut_ref[...], eye, (((0,), (0,)), ((), ())),
            preferred_element_type=jnp.float32)
        e0_ref[N_USERS:N, :] = lax.dot_general(
            it_ref[...], eye, (((0,), (0,)), ((), ())),
            preferred_element_type=jnp.float32)

    cur = jnp.where(l == 0, e0_ref[...], c1_ref[...])
    acc = jnp.dot(a_ref[...], cur, preferred_element_type=jnp.float32)
    nrm = jnp.sqrt(jnp.sum(acc * acc, axis=1, keepdims=True))
    c = acc / jnp.maximum(nrm, EPS)

    @pl.when(l == 0)
    def _():
        c1_ref[row, :] = c

    @pl.when(l == 1)
    def _():
        mean = (e0_ref[row, :] + c1_ref[row, :] + c) * (1.0 / 3.0)
        mean_ref[row, :] = mean
        pad_ref[:, 0:D] = mean

    @pl.when((l == 1) & (m == NM - 1))
    def _():
        eye = jnp.eye(D, dtype=jnp.float32)
        out_ref[...] = lax.dot_general(eye, mean_ref[...],
                                       (((1,), (1,)), ((), ())),
                                       preferred_element_type=jnp.float32)


def _propagate(a, ut, it, interpret=False):
    return pl.pallas_call(
        _gcn_body,
        grid=(2, NM),
        in_specs=[
            pl.BlockSpec((BM, N), lambda l, m: (m, 0)),
            pl.BlockSpec((D, N_USERS), lambda l, m: (0, 0)),
            pl.BlockSpec((D, N_ITEMS), lambda l, m: (0, 0)),
        ],
        out_specs=[
            pl.BlockSpec((D, N), lambda l, m: (0, 0)),
            pl.BlockSpec((BM, 128), lambda l, m: (m, 0)),
        ],
        out_shape=[
            jax.ShapeDtypeStruct((D, N), jnp.float32),
            jax.ShapeDtypeStruct((N, 128), jnp.float32),
        ],
        scratch_shapes=[
            pltpu.VMEM((N, D), jnp.float32),
            pltpu.VMEM((N, D), jnp.float32),
            pltpu.VMEM((N, D), jnp.float32),
        ],
        interpret=interpret,
    )(a, ut, it)


B = 4096
NW = 32
BPW = B // NW

_SC_MESH = plsc.VectorSubcoreMesh(core_axis_name="c", subcore_axis_name="s")


@functools.partial(
    pl.kernel,
    mesh=_SC_MESH,
    out_type=jax.ShapeDtypeStruct((3 * B, 128), jnp.float32),
    scratch_types=[
        pltpu.VMEM((BPW,), jnp.int32),
        pltpu.VMEM((BPW,), jnp.int32),
        pltpu.VMEM((BPW,), jnp.int32),
        pltpu.VMEM((BPW, 128), jnp.float32),
        pltpu.VMEM((BPW, 128), jnp.float32),
        pltpu.VMEM((BPW, 128), jnp.float32),
        pltpu.SemaphoreType.DMA,
        pltpu.SemaphoreType.DMA,
    ],
)
def _sc_gather(emb_hbm, uid_hbm, pid_hbm, nid_hbm, out_hbm,
               uidx, pidx, nidx, urows, prows, nrows, sem, wsem):
    wid = lax.axis_index("s") * 2 + lax.axis_index("c")
    base = wid * BPW
    iu = pltpu.async_copy(uid_hbm.at[pl.ds(base, BPW)], uidx, wsem)
    ip = pltpu.async_copy(pid_hbm.at[pl.ds(base, BPW)], pidx, wsem)
    inn = pltpu.async_copy(nid_hbm.at[pl.ds(base, BPW)], nidx, wsem)
    iu.wait()
    cu = pltpu.async_copy(emb_hbm.at[uidx], urows, sem)
    ip.wait()
    inn.wait()
    for j in range(BPW // 16):
        sl = pl.ds(j * 16, 16)
        pidx[sl] = pidx[sl] + N_USERS
        nidx[sl] = nidx[sl] + N_USERS
    cp = pltpu.async_copy(emb_hbm.at[pidx], prows, sem)
    cn = pltpu.async_copy(emb_hbm.at[nidx], nrows, sem)
    cu.wait()
    wu = pltpu.async_copy(urows, out_hbm.at[pl.ds(base, BPW)], wsem)
    cp.wait()
    wp = pltpu.async_copy(prows, out_hbm.at[pl.ds(B + base, BPW)], wsem)
    cn.wait()
    wn = pltpu.async_copy(nrows, out_hbm.at[pl.ds(2 * B + base, BPW)], wsem)
    wu.wait()
    wp.wait()
    wn.wait()


NC = 4
CB = B // NC


def _loss_body(u_ref, p_ref, n_ref, out_ref, acc_ref):
    i = pl.program_id(0)
    u = u_ref[:, 0:D]
    diff = (jnp.sum(u * p_ref[:, 0:D], axis=1)
            - jnp.sum(u * n_ref[:, 0:D], axis=1))
    part = jnp.sum(jnp.log(jax.nn.sigmoid(diff)))

    @pl.when(i == 0)
    def _():
        acc_ref[0] = part

    @pl.when(i > 0)
    def _():
        acc_ref[0] += part

    @pl.when(i == NC - 1)
    def _():
        out_ref[0, 0] = -acc_ref[0] * (1.0 / B)


def _loss(rows):
    out = pl.pallas_call(
        _loss_body,
        grid=(NC,),
        in_specs=[
            pl.BlockSpec((CB, 128), lambda i: (i, 0)),
            pl.BlockSpec((CB, 128), lambda i: (i + NC, 0)),
            pl.BlockSpec((CB, 128), lambda i: (i + 2 * NC, 0)),
        ],
        out_specs=pl.BlockSpec(memory_space=pltpu.SMEM),
        out_shape=jax.ShapeDtypeStruct((1, 1), jnp.float32),
        scratch_shapes=[pltpu.SMEM((1,), jnp.float32)],
    )(rows, rows, rows)
    return out[0, 0]


def kernel(user_emb, item_emb, edge_index, user_id, pos_item, neg_item):
    out_t, pad = _propagate(edge_index, user_emb.T, item_emb.T)
    all_emb = out_t.T
    rows = _sc_gather(pad, user_id, pos_item, neg_item)
    rec_loss = _loss(rows)
    return (rec_loss, all_emb)
